# baseline (speedup 1.0000x reference)
import functools

import jax
import jax.numpy as jnp
from jax import lax
from jax.experimental import pallas as pl
from jax.experimental.pallas import tpu as pltpu
from jax.experimental.pallas import tpu_sc as plsc

_NUM_CORES = 2
_NUM_SUBCORES = 16
_NUM_WORKERS = _NUM_CORES * _NUM_SUBCORES
_LANES = 16


def _tc_cols_body(idx_ref, table_ref, c0_ref, c1_ref, win, sem):
    d = win.shape[0]
    outs = (c0_ref, c1_ref)
    for k in range(2):
        base = (idx_ref[k] // 128) * 128
        cp = pltpu.make_async_copy(
            table_ref.at[:, pl.ds(base, 128)], win, sem)
        cp.start()
        cp.wait()
        off = idx_ref[k] % 128
        lane = jax.lax.broadcasted_iota(jnp.int32, (d, 128), 1)
        col = jnp.sum(
            jnp.where(lane == off, win[...], 0.0), axis=1, keepdims=True)
        outs[k][...] = jnp.broadcast_to(col, (d, _LANES))


def _sc_bcast_body(chunk, d, c0_hbm, c1_hbm, o0_hbm, o1_hbm,
                   c0_v, c1_v, buf0, buf1, sem0, sem1):
    wid = lax.axis_index("s") * _NUM_CORES + lax.axis_index("c")
    base = wid * chunk

    pltpu.sync_copy(c0_hbm, c0_v)
    pltpu.sync_copy(c1_hbm, c1_v)

    nlane = chunk // _LANES
    for i in range(d):
        v0 = c0_v[i, pl.ds(0, _LANES)]
        v1 = c1_v[i, pl.ds(0, _LANES)]

        def fill(c, carry, i=i, v0=v0, v1=v1):
            buf0[i, pl.ds(c * _LANES, _LANES)] = v0
            buf1[i, pl.ds(c * _LANES, _LANES)] = v1
            return carry

        lax.fori_loop(0, nlane, fill, 0)

    cp0 = pltpu.make_async_copy(buf0, o0_hbm.at[:, pl.ds(base, chunk)], sem0)
    cp1 = pltpu.make_async_copy(buf1, o1_hbm.at[:, pl.ds(base, chunk)], sem1)
    cp0.start()
    cp1.start()
    cp0.wait()
    cp1.wait()


def kernel(rel, table, indices):
    batch = rel.shape[0]
    d = table.shape[1]
    chunk = batch // _NUM_WORKERS
    table_t = table.T

    cols_sds = jax.ShapeDtypeStruct((d, _LANES), jnp.float32)
    c0, c1 = pl.pallas_call(
        _tc_cols_body,
        grid_spec=pltpu.PrefetchScalarGridSpec(
            num_scalar_prefetch=1,
            grid=(1,),
            in_specs=[pl.BlockSpec(memory_space=pl.ANY)],
            out_specs=[
                pl.BlockSpec((d, _LANES), lambda i, idx: (0, 0)),
                pl.BlockSpec((d, _LANES), lambda i, idx: (0, 0)),
            ],
            scratch_shapes=[
                pltpu.VMEM((d, 128), jnp.float32),
                pltpu.SemaphoreType.DMA,
            ],
        ),
        out_shape=[cols_sds, cols_sds],
    )(indices.astype(jnp.int32), table_t)

    mesh = plsc.VectorSubcoreMesh(core_axis_name="c", subcore_axis_name="s")
    out_sds = jax.ShapeDtypeStruct((d, batch), jnp.float32)
    sc_call = pl.kernel(
        functools.partial(_sc_bcast_body, chunk, d),
        out_type=[out_sds, out_sds],
        mesh=mesh,
        scratch_types=[
            pltpu.VMEM((d, _LANES), jnp.float32),
            pltpu.VMEM((d, _LANES), jnp.float32),
            pltpu.VMEM((d, chunk), jnp.float32),
            pltpu.VMEM((d, chunk), jnp.float32),
            pltpu.SemaphoreType.DMA,
            pltpu.SemaphoreType.DMA,
        ],
    )
    o0, o1 = sc_call(c0, c1)
    return (o0.T, o1.T)


# speedup vs baseline: 6.2201x; 6.2201x over previous
import jax
import jax.numpy as jnp
from jax.experimental import pallas as pl
from jax.experimental.pallas import tpu as pltpu

_BLOCK_B = 4096


def _tc_body(idx_ref, table_ref, o0_ref, o1_ref, cols, win, sem):
    i = pl.program_id(0)

    @pl.when(i == 0)
    def _fetch_cols():
        d = win.shape[1]
        cps = [
            pltpu.make_async_copy(
                table_ref.at[:, pl.ds((idx_ref[k] // 128) * 128, 128)],
                win.at[k], sem)
            for k in range(2)
        ]
        for cp in cps:
            cp.start()
        for cp in cps:
            cp.wait()
        lane = jax.lax.broadcasted_iota(jnp.int32, (d, 128), 1)
        for k in range(2):
            off = idx_ref[k] % 128
            colk = jnp.sum(
                jnp.where(lane == off, win[k], 0.0), axis=1, keepdims=True)
            cols[:, pl.ds(k, 1)] = colk

    o0_ref[...] = jnp.broadcast_to(cols[:, 0:1], o0_ref.shape)
    o1_ref[...] = jnp.broadcast_to(cols[:, 1:2], o1_ref.shape)


def kernel(rel, table, indices):
    batch = rel.shape[0]
    d = table.shape[1]
    table_t = table.T
    grid = (batch // _BLOCK_B,)
    out_sds = jax.ShapeDtypeStruct((d, batch), jnp.float32)
    o0, o1 = pl.pallas_call(
        _tc_body,
        grid_spec=pltpu.PrefetchScalarGridSpec(
            num_scalar_prefetch=1,
            grid=grid,
            in_specs=[pl.BlockSpec(memory_space=pl.ANY)],
            out_specs=[
                pl.BlockSpec((d, _BLOCK_B), lambda i, idx: (0, i)),
                pl.BlockSpec((d, _BLOCK_B), lambda i, idx: (0, i)),
            ],
            scratch_shapes=[
                pltpu.VMEM((d, 2), jnp.float32),
                pltpu.VMEM((2, d, 128), jnp.float32),
                pltpu.SemaphoreType.DMA,
            ],
        ),
        out_shape=[out_sds, out_sds],
    )(indices.astype(jnp.int32), table_t)
    return (o0.T, o1.T)


# speedup vs baseline: 6.4435x; 1.0359x over previous
import jax
import jax.numpy as jnp
from jax.experimental import pallas as pl
from jax.experimental.pallas import tpu as pltpu

_BLOCK_B = 8192


def _tc_body(idx_ref, table_ref, o0_ref, o1_ref, cols, win, sem):
    i = pl.program_id(0)

    @pl.when(i == 0)
    def _fetch_cols():
        d = win.shape[1]
        cps = [
            pltpu.make_async_copy(
                table_ref.at[:, pl.ds((idx_ref[k] // 128) * 128, 128)],
                win.at[k], sem)
            for k in range(2)
        ]
        for cp in cps:
            cp.start()
        for cp in cps:
            cp.wait()
        lane = jax.lax.broadcasted_iota(jnp.int32, (d, 128), 1)
        for k in range(2):
            off = idx_ref[k] % 128
            colk = jnp.sum(
                jnp.where(lane == off, win[k], 0.0), axis=1, keepdims=True)
            cols[:, pl.ds(k, 1)] = colk

    o0_ref[...] = jnp.broadcast_to(cols[:, 0:1], o0_ref.shape)
    o1_ref[...] = jnp.broadcast_to(cols[:, 1:2], o1_ref.shape)


def kernel(rel, table, indices):
    batch = rel.shape[0]
    d = table.shape[1]
    table_t = table.T
    grid = (batch // _BLOCK_B,)
    out_sds = jax.ShapeDtypeStruct((d, batch), jnp.float32)
    o0, o1 = pl.pallas_call(
        _tc_body,
        grid_spec=pltpu.PrefetchScalarGridSpec(
            num_scalar_prefetch=1,
            grid=grid,
            in_specs=[pl.BlockSpec(memory_space=pl.ANY)],
            out_specs=[
                pl.BlockSpec((d, _BLOCK_B), lambda i, idx: (0, i)),
                pl.BlockSpec((d, _BLOCK_B), lambda i, idx: (0, i)),
            ],
            scratch_shapes=[
                pltpu.VMEM((d, 2), jnp.float32),
                pltpu.VMEM((2, d, 128), jnp.float32),
                pltpu.SemaphoreType.DMA,
            ],
        ),
        out_shape=[out_sds, out_sds],
    )(indices.astype(jnp.int32), table_t)
    return (o0.T, o1.T)
